# baseline (speedup 1.0000x reference)
import functools

import numpy as np
import jax
import jax.numpy as jnp
from jax.experimental import pallas as pl
from jax.experimental.pallas import tpu as pltpu
from jax.sharding import Mesh, PartitionSpec as P

_C = 128
_NBLK = 9


def _relu(a):
    return jnp.maximum(a, 0.0)


def _rup8(n):
    return ((n + 7) // 8) * 8


def _rup16(n):
    return ((n + 15) // 16) * 16


def _conv_body(x_ref, c0_ref, rhs0_ref, dwW_ref, db8_ref, W2_ref, b2_ref,
               s28_ref, t28_ref, edge8_ref, out_ref,
               xa0_ref, xa1_ref, yscr_ref, zscr_ref,
               *, NB, L0, lengths):
    f32 = jnp.float32
    bf16 = jnp.bfloat16
    wvr = c0_ref[0:1, :]
    cbr = c0_ref[1:2, :]
    s2r = c0_ref[2:3, :]
    t2r = c0_ref[3:4, :]
    e0r = c0_ref[5:6, :]
    srow = c0_ref[6:7, :]
    d0 = srow[:, 0:1]
    d1 = srow[:, 1:2]
    d2 = srow[:, 2:3]
    db0 = srow[:, 3:4]
    s1v = srow[:, 4:5]
    t1v = srow[:, 5:6]

    x = x_ref[...]
    zc = jnp.zeros((NB, 1), f32)
    xl = jnp.concatenate([x[:, 1:], zc], axis=1)
    xr = jnp.concatenate([zc, x[:, :-1]], axis=1)
    y0 = _relu(xr * d0 + x * d1 + xl * d2 + db0) * s1v + t1v
    yscr_ref[...] = jnp.transpose(y0)
    K0 = L0 // 2
    ya = yscr_ref[0:2 * K0:2]
    yb = yscr_ref[1:2 * K0:2]
    mx = jnp.maximum(ya, yb)
    mn = jnp.minimum(ya, yb)
    ones = jnp.ones((K0, 1), f32)
    lhs = jnp.concatenate([mx, mn, ones], axis=1).astype(bf16)
    e0b = e0r.astype(bf16)
    zrow = jnp.zeros((1, _C), bf16)
    for i in range(NB):
        zp = jnp.dot(lhs, rhs0_ref[i], preferred_element_type=f32)
        zi = ((s2r * _relu(zp) + t2r)).astype(bf16)
        if L0 % 2:
            yl = yscr_ref[L0 - 1:L0, i:i + 1]
            zl = s2r * _relu(wvr * yl + cbr) + t2r
            last = jnp.maximum(zl, e0r).astype(bf16)
        else:
            last = e0b
        xa0_ref[i, 0:1] = zrow
        xa0_ref[i, 1:2] = e0b
        xa0_ref[i, 2:2 + K0] = zi
        xa0_ref[i, 2 + K0:3 + K0] = last
        xa0_ref[i, 3 + K0:4 + K0] = zrow

    srcA, dstA = xa0_ref, xa1_ref
    Rp = _rup16(lengths[1] + 2)
    zrow3 = jnp.zeros((NB, 1, _C), bf16)
    for j in range(_NBLK - 1):
        L = lengths[j + 1]
        w3 = dwW_ref[j]
        Y = _relu(srcA[:, 0:Rp] * w3[0:1, :]
                  + srcA[:, 1:Rp + 1] * w3[1:2, :]
                  + srcA[:, 2:Rp + 2] * w3[2:3, :] + db8_ref[j])
        Z = jnp.dot(Y.reshape(NB * Rp, _C), W2_ref[j],
                    preferred_element_type=f32)
        Z = Z.reshape(NB, Rp, _C) + b2_ref[j]
        zscr_ref[:, 0:Rp, :] = _relu(Z) * s28_ref[j] + t28_ref[j]

        K = L // 2
        mxz = jnp.maximum(zscr_ref[:, 0:2 * K:2, :],
                          zscr_ref[:, 1:2 * K:2, :])
        er3 = jnp.broadcast_to(edge8_ref[j].reshape(1, 1, _C), (NB, 1, _C))
        if L % 2:
            last3 = jnp.maximum(zscr_ref[:, L - 1:L, :], er3)
        else:
            last3 = er3
        if j < _NBLK - 2:
            dstA[:, 0:1] = zrow3
            dstA[:, 1:2] = er3.astype(bf16)
            dstA[:, 2:2 + K] = mxz.astype(bf16)
            dstA[:, 2 + K:3 + K] = last3.astype(bf16)
            dstA[:, 3 + K:4 + K] = zrow3
            srcA, dstA = dstA, srcA
            Rp = _rup16(K + 4)
        else:
            out_ref[...] = jnp.concatenate([er3, mxz, last3], axis=1)


def _head_body(fd_ref, fm_ref, w0_ref, b0_ref, w1_ref, b1_ref,
               wa_ref, wb_ref, bo1_ref, w2_ref, bo2_ref, out_ref, *, TB):
    f32 = jnp.float32
    f = jnp.concatenate([fd_ref[...], fm_ref[...]], axis=0)
    h = jnp.tanh(jnp.dot(f.astype(jnp.bfloat16), w0_ref[...],
                         preferred_element_type=f32) + b0_ref[...])
    h = jnp.tanh(jnp.dot(h.astype(jnp.bfloat16), w1_ref[...],
                         preferred_element_type=f32) + b1_ref[...])
    o = (jnp.dot(h[:TB].astype(jnp.bfloat16), wa_ref[...],
                 preferred_element_type=f32)
         + jnp.dot(h[TB:].astype(jnp.bfloat16), wb_ref[...],
                   preferred_element_type=f32)
         + bo1_ref[...])
    out_ref[...] = jnp.dot(o, w2_ref[...],
                           preferred_element_type=f32) + bo2_ref[...]


def kernel(data, meta, dw_w, dw_b, s1, t1, pw_w, pw_b, s2, t2,
           w0_eff, b0, w1, b1, wo1a, wo1b, bo1, wo2, bo2):
    args = (data, meta, dw_w, dw_b, s1, t1, pw_w, pw_b, s2, t2,
            w0_eff, b0, w1, b1, wo1a, wo1b, bo1, wo2, bo2)
    devs = jax.devices()
    n_dev = 2 if (len(devs) >= 2 and data.shape[0] % 512 == 0) else 1
    if n_dev == 1:
        return _impl(*args)
    mesh = Mesh(np.asarray(devs[:n_dev]), ('b',))
    specs = (P('b'), P('b')) + (P(),) * 17
    return jax.shard_map(_impl, mesh=mesh, in_specs=specs,
                         out_specs=P('b'), check_vma=False)(*args)


def _impl(data, meta, dw_w, dw_b, s1, t1, pw_w, pw_b, s2, t2,
          w0_eff, b0, w1, b1, wo1a, wo1b, bo1, wo2, bo2):
    f32 = jnp.float32
    bf16 = jnp.bfloat16
    N, _, L0 = data.shape
    B = 2 * N
    lengths = [L0]
    L = L0
    for _ in range(_NBLK):
        L = (L + 2) // 2 + 1
        lengths.append(L)
    T = lengths[-1]

    x_all = jnp.concatenate([data, meta], axis=0).reshape(B, L0).astype(f32)

    k0 = _relu(dw_b[0]) * s1[0] + t1[0]
    cb = (jnp.dot(k0, pw_w[0]) - k0[0, 0] * pw_w[0, 0:1, :] + pw_b[0])
    wv = pw_w[0, 0:1, :]
    edge0 = _relu(pw_b[0]) * s2[0] + t2[0]
    gsel = (wv * s2[0] >= 0).astype(f32)
    srow = jnp.pad(
        jnp.stack([dw_w[0, 0, 0], dw_w[0, 1, 0], dw_w[0, 2, 0],
                   dw_b[0, 0, 0], s1[0, 0, 0], t1[0, 0, 0]]).reshape(1, 6),
        ((0, 0), (0, _C - 6)))
    c0 = jnp.concatenate(
        [wv, cb, s2[0], t2[0], gsel, edge0, srow, jnp.zeros((1, _C), f32)],
        axis=0)

    dwW = dw_w[1:].astype(bf16)
    db8 = dw_b[1:].astype(bf16)
    W2 = (jnp.swapaxes(s1[1:], 1, 2) * pw_w[1:]).astype(bf16)
    b2 = jnp.einsum('boc,bcd->bod', t1[1:], pw_w[1:]) + pw_b[1:]
    edge8 = _relu(pw_b[1:]) * s2[1:] + t2[1:]
    s28 = s2[1:]
    t28 = t2[1:]

    NB = 8
    while B % NB:
        NB //= 2
    wp = (wv * gsel)[0]
    wm = (wv * (1.0 - gsel))[0]
    rhs_rows = []
    for i in range(NB):
        r = jnp.zeros((2 * NB + 1, _C), f32)
        r = r.at[i].set(wp).at[NB + i].set(wm).at[2 * NB].set(cb[0])
        rhs_rows.append(r)
    rhs0 = jnp.stack(rhs_rows).astype(bf16)
    R1 = _rup16(lengths[1] + 2)

    def _w(a):
        return pl.BlockSpec(a.shape, lambda i, nd=a.ndim: (0,) * nd)

    body = functools.partial(_conv_body, NB=NB, L0=L0, lengths=tuple(lengths))
    feats = pl.pallas_call(
        body,
        out_shape=jax.ShapeDtypeStruct((B, T, _C), f32),
        grid=(B // NB,),
        in_specs=[
            pl.BlockSpec((NB, L0), lambda i: (i, 0)),
            _w(c0), _w(rhs0), _w(dwW), _w(db8), _w(W2), _w(b2),
            _w(s28), _w(t28), _w(edge8),
        ],
        out_specs=pl.BlockSpec((NB, T, _C), lambda i: (i, 0, 0)),
        scratch_shapes=[pltpu.VMEM((NB, R1 + 16, _C), bf16),
                        pltpu.VMEM((NB, R1 + 16, _C), bf16),
                        pltpu.VMEM((L0, NB), f32),
                        pltpu.VMEM((NB, R1, _C), f32)],
        compiler_params=pltpu.CompilerParams(
            dimension_semantics=("parallel",),
            vmem_limit_bytes=64 * 1024 * 1024),
    )(x_all, c0, rhs0, dwW, db8, W2, b2, s28, t28, edge8)

    feat2d = feats.reshape(B, T * _C)

    TB = 256 if N % 256 == 0 else N
    w0b = w0_eff.astype(bf16)
    w1b = w1.astype(bf16)
    wab = wo1a.astype(bf16)
    wbb = wo1b.astype(bf16)
    hbody = functools.partial(_head_body, TB=TB)
    out = pl.pallas_call(
        hbody,
        out_shape=jax.ShapeDtypeStruct((N, wo2.shape[1]), f32),
        grid=(N // TB,),
        in_specs=[
            pl.BlockSpec((TB, T * _C), lambda i: (i, 0)),
            pl.BlockSpec((TB, T * _C), lambda i, o=N // TB: (i + o, 0)),
            _w(w0b), _w(b0), _w(w1b), _w(b1),
            _w(wab), _w(wbb), _w(bo1), _w(wo2), _w(bo2),
        ],
        out_specs=pl.BlockSpec((TB, wo2.shape[1]), lambda i: (i, 0)),
        compiler_params=pltpu.CompilerParams(
            dimension_semantics=("parallel",)),
    )(feat2d, feat2d, w0b, b0, w1b, b1, wab, wbb, bo1, wo2, bo2)
    return out


# speedup vs baseline: 1.5111x; 1.5111x over previous
import functools

import numpy as np
import jax
import jax.numpy as jnp
from jax.experimental import pallas as pl
from jax.experimental.pallas import tpu as pltpu
from jax.sharding import Mesh, PartitionSpec as P

_C = 128
_NBLK = 9


def _relu(a):
    return jnp.maximum(a, 0.0)


def _rup8(n):
    return ((n + 7) // 8) * 8


def _rup16(n):
    return ((n + 15) // 16) * 16


def _conv_body(x_ref, c0_ref, dwW_ref, db8_ref, W2_ref, b2_ref,
               s28_ref, t28_ref, edge8_ref, out_ref,
               xchunk_ref, yscr_ref, zscr_ref, *, NB, L0, lengths):
    f32 = jnp.float32
    bf16 = jnp.bfloat16
    wvr = c0_ref[0:1, :]
    cbr = c0_ref[1:2, :]
    s2r = c0_ref[2:3, :]
    t2r = c0_ref[3:4, :]
    gselr = c0_ref[4:5, :]
    e0r = c0_ref[5:6, :]
    srow = c0_ref[6:7, :]
    d0 = srow[:, 0:1]
    d1 = srow[:, 1:2]
    d2 = srow[:, 2:3]
    db0 = srow[:, 3:4]
    s1v = srow[:, 4:5]
    t1v = srow[:, 5:6]

    x = x_ref[...]
    zc = jnp.zeros((NB, 1), f32)
    xl = jnp.concatenate([x[:, 1:], zc], axis=1)
    xr = jnp.concatenate([zc, x[:, :-1]], axis=1)
    y0 = _relu(xr * d0 + x * d1 + xl * d2 + db0) * s1v + t1v
    yscr_ref[...] = jnp.transpose(y0)
    K0 = L0 // 2
    ya = yscr_ref[0:2 * K0:2]
    yb = yscr_ref[1:2 * K0:2]
    mx = jnp.maximum(ya, yb)
    mn = jnp.minimum(ya, yb)
    gsel = gselr > 0.5

    P1 = lengths[1]
    R1 = _rup16(P1 + 2)
    tail0 = R1 - (P1 + 2) + 1
    z1 = jnp.zeros((1, _C), f32)
    ztail0 = jnp.zeros((tail0, _C), f32)
    for i in range(NB):
        g = jnp.where(gsel, mx[:, i:i + 1], mn[:, i:i + 1])
        zi = s2r * _relu(wvr * g + cbr) + t2r
        if L0 % 2:
            yl = yscr_ref[L0 - 1:L0, i:i + 1]
            zl = s2r * _relu(wvr * yl + cbr) + t2r
            last = jnp.maximum(zl, e0r)
        else:
            last = e0r
        chunk = jnp.concatenate([z1, e0r, zi, last, ztail0], axis=0)
        xchunk_ref[i] = chunk.astype(bf16)

    X = xchunk_ref[...]
    Rp = R1
    for j in range(_NBLK - 1):
        L = lengths[j + 1]
        w3 = dwW_ref[j]
        Xs1 = jnp.concatenate(
            [X[:, 1:], jnp.zeros((NB, 1, _C), bf16)], axis=1)
        Xs2 = jnp.concatenate(
            [X[:, 2:], jnp.zeros((NB, 2, _C), bf16)], axis=1)
        Y = _relu(X * w3[0:1, :] + Xs1 * w3[1:2, :] + Xs2 * w3[2:3, :]
                  + db8_ref[j])
        Yb = Y.reshape(NB * Rp, _C)
        Z = jnp.dot(Yb, W2_ref[j], preferred_element_type=f32)
        Z = Z.reshape(NB, Rp, _C) + b2_ref[j]
        zscr_ref[:, 0:Rp, :] = _relu(Z) * s28_ref[j] + t28_ref[j]

        K = L // 2
        mxz = jnp.maximum(zscr_ref[:, 0:2 * K:2, :],
                          zscr_ref[:, 1:2 * K:2, :])
        er3 = jnp.broadcast_to(edge8_ref[j].reshape(1, 1, _C), (NB, 1, _C))
        if L % 2:
            last3 = jnp.maximum(zscr_ref[:, L - 1:L, :], er3)
        else:
            last3 = er3
        if j < _NBLK - 2:
            Pn = K + 2
            Rn = _rup16(Pn + 2)
            tl = Rn - (Pn + 2) + 1
            X = jnp.concatenate(
                [jnp.zeros((NB, 1, _C), bf16), er3.astype(bf16),
                 mxz.astype(bf16), last3.astype(bf16),
                 jnp.zeros((NB, tl, _C), bf16)], axis=1)
            Rp = Rn
        else:
            out_ref[...] = jnp.concatenate([er3, mxz, last3], axis=1)


def _head_body(fd_ref, fm_ref, w0_ref, b0_ref, w1_ref, b1_ref,
               wa_ref, wb_ref, bo1_ref, w2_ref, bo2_ref, out_ref, *, TB):
    f32 = jnp.float32
    f = jnp.concatenate([fd_ref[...], fm_ref[...]], axis=0)
    h = jnp.tanh(jnp.dot(f.astype(jnp.bfloat16), w0_ref[...],
                         preferred_element_type=f32) + b0_ref[...])
    h = jnp.tanh(jnp.dot(h.astype(jnp.bfloat16), w1_ref[...],
                         preferred_element_type=f32) + b1_ref[...])
    o = (jnp.dot(h[:TB].astype(jnp.bfloat16), wa_ref[...],
                 preferred_element_type=f32)
         + jnp.dot(h[TB:].astype(jnp.bfloat16), wb_ref[...],
                   preferred_element_type=f32)
         + bo1_ref[...])
    out_ref[...] = jnp.dot(o, w2_ref[...],
                           preferred_element_type=f32) + bo2_ref[...]


def kernel(data, meta, dw_w, dw_b, s1, t1, pw_w, pw_b, s2, t2,
           w0_eff, b0, w1, b1, wo1a, wo1b, bo1, wo2, bo2):
    args = (data, meta, dw_w, dw_b, s1, t1, pw_w, pw_b, s2, t2,
            w0_eff, b0, w1, b1, wo1a, wo1b, bo1, wo2, bo2)
    devs = jax.devices()
    n_dev = 2 if (len(devs) >= 2 and data.shape[0] % 512 == 0) else 1
    if n_dev == 1:
        return _impl(*args)
    mesh = Mesh(np.asarray(devs[:n_dev]), ('b',))
    specs = (P('b'), P('b')) + (P(),) * 17
    return jax.shard_map(_impl, mesh=mesh, in_specs=specs,
                         out_specs=P('b'), check_vma=False)(*args)


def _impl(data, meta, dw_w, dw_b, s1, t1, pw_w, pw_b, s2, t2,
          w0_eff, b0, w1, b1, wo1a, wo1b, bo1, wo2, bo2):
    f32 = jnp.float32
    bf16 = jnp.bfloat16
    N, _, L0 = data.shape
    B = 2 * N
    lengths = [L0]
    L = L0
    for _ in range(_NBLK):
        L = (L + 2) // 2 + 1
        lengths.append(L)
    T = lengths[-1]

    x_all = jnp.concatenate([data, meta], axis=0).reshape(B, L0).astype(f32)

    k0 = _relu(dw_b[0]) * s1[0] + t1[0]
    cb = (jnp.dot(k0, pw_w[0]) - k0[0, 0] * pw_w[0, 0:1, :] + pw_b[0])
    wv = pw_w[0, 0:1, :]
    edge0 = _relu(pw_b[0]) * s2[0] + t2[0]
    gsel = (wv * s2[0] >= 0).astype(f32)
    srow = jnp.pad(
        jnp.stack([dw_w[0, 0, 0], dw_w[0, 1, 0], dw_w[0, 2, 0],
                   dw_b[0, 0, 0], s1[0, 0, 0], t1[0, 0, 0]]).reshape(1, 6),
        ((0, 0), (0, _C - 6)))
    c0 = jnp.concatenate(
        [wv, cb, s2[0], t2[0], gsel, edge0, srow, jnp.zeros((1, _C), f32)],
        axis=0)

    dwW = dw_w[1:].astype(bf16)
    db8 = dw_b[1:].astype(bf16)
    W2 = (jnp.swapaxes(s1[1:], 1, 2) * pw_w[1:]).astype(bf16)
    b2 = jnp.einsum('boc,bcd->bod', t1[1:], pw_w[1:]) + pw_b[1:]
    edge8 = _relu(pw_b[1:]) * s2[1:] + t2[1:]
    s28 = s2[1:]
    t28 = t2[1:]

    NB = 8
    while B % NB:
        NB //= 2
    R1 = _rup16(lengths[1] + 2)

    def _w(a):
        return pl.BlockSpec(a.shape, lambda i, nd=a.ndim: (0,) * nd)

    body = functools.partial(_conv_body, NB=NB, L0=L0, lengths=tuple(lengths))
    feats = pl.pallas_call(
        body,
        out_shape=jax.ShapeDtypeStruct((B, T, _C), f32),
        grid=(B // NB,),
        in_specs=[
            pl.BlockSpec((NB, L0), lambda i: (i, 0)),
            _w(c0), _w(dwW), _w(db8), _w(W2), _w(b2),
            _w(s28), _w(t28), _w(edge8),
        ],
        out_specs=pl.BlockSpec((NB, T, _C), lambda i: (i, 0, 0)),
        scratch_shapes=[pltpu.VMEM((NB, R1, _C), bf16),
                        pltpu.VMEM((L0, NB), f32),
                        pltpu.VMEM((NB, R1, _C), f32)],
        compiler_params=pltpu.CompilerParams(
            dimension_semantics=("parallel",),
            vmem_limit_bytes=64 * 1024 * 1024),
    )(x_all, c0, dwW, db8, W2, b2, s28, t28, edge8)

    feat2d = feats.reshape(B, T * _C)

    TB = 256 if N % 256 == 0 else N
    w0b = w0_eff.astype(bf16)
    w1b = w1.astype(bf16)
    wab = wo1a.astype(bf16)
    wbb = wo1b.astype(bf16)
    hbody = functools.partial(_head_body, TB=TB)
    out = pl.pallas_call(
        hbody,
        out_shape=jax.ShapeDtypeStruct((N, wo2.shape[1]), f32),
        grid=(N // TB,),
        in_specs=[
            pl.BlockSpec((TB, T * _C), lambda i: (i, 0)),
            pl.BlockSpec((TB, T * _C), lambda i, o=N // TB: (i + o, 0)),
            _w(w0b), _w(b0), _w(w1b), _w(b1),
            _w(wab), _w(wbb), _w(bo1), _w(wo2), _w(bo2),
        ],
        out_specs=pl.BlockSpec((TB, wo2.shape[1]), lambda i: (i, 0)),
        compiler_params=pltpu.CompilerParams(
            dimension_semantics=("parallel",)),
    )(feat2d, feat2d, w0b, b0, w1b, b1, wab, wbb, bo1, wo2, bo2)
    return out


# speedup vs baseline: 1.5998x; 1.0587x over previous
import functools

import numpy as np
import jax
import jax.numpy as jnp
from jax.experimental import pallas as pl
from jax.experimental.pallas import tpu as pltpu
from jax.sharding import Mesh, PartitionSpec as P

_C = 128
_NBLK = 9


def _relu(a):
    return jnp.maximum(a, 0.0)


def _rup8(n):
    return ((n + 7) // 8) * 8


def _rup16(n):
    return ((n + 15) // 16) * 16


def _conv_body(x_ref, c0_ref, dwW_ref, db8_ref, W2_ref, b2_ref,
               s28_ref, t28_ref, edge8_ref, out_ref,
               xchunk_ref, yscr_ref, zscr_ref, *, NB, L0, lengths):
    f32 = jnp.float32
    bf16 = jnp.bfloat16
    wvr = c0_ref[0:1, :]
    cbr = c0_ref[1:2, :]
    s2r = c0_ref[2:3, :]
    t2r = c0_ref[3:4, :]
    gselr = c0_ref[4:5, :]
    e0r = c0_ref[5:6, :]
    srow = c0_ref[6:7, :]
    d0 = srow[:, 0:1]
    d1 = srow[:, 1:2]
    d2 = srow[:, 2:3]
    db0 = srow[:, 3:4]
    s1v = srow[:, 4:5]
    t1v = srow[:, 5:6]

    x = x_ref[...]
    zc = jnp.zeros((NB, 1), f32)
    xl = jnp.concatenate([x[:, 1:], zc], axis=1)
    xr = jnp.concatenate([zc, x[:, :-1]], axis=1)
    y0 = _relu(xr * d0 + x * d1 + xl * d2 + db0) * s1v + t1v
    yscr_ref[...] = jnp.transpose(y0)
    K0 = L0 // 2
    ya = yscr_ref[0:2 * K0:2]
    yb = yscr_ref[1:2 * K0:2]
    mx = jnp.maximum(ya, yb)
    mn = jnp.minimum(ya, yb)
    gsel = gselr > 0.5

    P1 = lengths[1]
    R1 = _rup16(P1 + 2)
    tail0 = R1 - (P1 + 2) + 1
    z1 = jnp.zeros((1, _C), f32)
    ztail0 = jnp.zeros((tail0, _C), f32)
    for i in range(NB):
        g = jnp.where(gsel, mx[:, i:i + 1], mn[:, i:i + 1])
        zi = s2r * _relu(wvr * g + cbr) + t2r
        if L0 % 2:
            yl = yscr_ref[L0 - 1:L0, i:i + 1]
            zl = s2r * _relu(wvr * yl + cbr) + t2r
            last = jnp.maximum(zl, e0r)
        else:
            last = e0r
        chunk = jnp.concatenate([z1, e0r, zi, last, ztail0], axis=0)
        xchunk_ref[i] = chunk.astype(bf16)

    X = xchunk_ref[...]
    Rp = R1
    for j in range(_NBLK - 1):
        L = lengths[j + 1]
        w3 = dwW_ref[j]
        Xs1 = jnp.concatenate(
            [X[:, 1:], jnp.zeros((NB, 1, _C), bf16)], axis=1)
        Xs2 = jnp.concatenate(
            [X[:, 2:], jnp.zeros((NB, 2, _C), bf16)], axis=1)
        Y = _relu(X * w3[0:1, :] + Xs1 * w3[1:2, :] + Xs2 * w3[2:3, :]
                  + db8_ref[j])
        Yb = Y.reshape(NB * Rp, _C)
        Z = jnp.dot(Yb, W2_ref[j], preferred_element_type=f32)
        Z = Z.reshape(NB, Rp, _C) + b2_ref[j]
        zscr_ref[:, 0:Rp, :] = _relu(Z) * s28_ref[j] + t28_ref[j]

        K = L // 2
        mxz = jnp.maximum(zscr_ref[:, 0:2 * K:2, :],
                          zscr_ref[:, 1:2 * K:2, :])
        er3 = jnp.broadcast_to(edge8_ref[j].reshape(1, 1, _C), (NB, 1, _C))
        if L % 2:
            last3 = jnp.maximum(zscr_ref[:, L - 1:L, :], er3)
        else:
            last3 = er3
        if j < _NBLK - 2:
            Pn = K + 2
            Rn = _rup16(Pn + 2)
            tl = Rn - (Pn + 2) + 1
            X = jnp.concatenate(
                [jnp.zeros((NB, 1, _C), bf16), er3.astype(bf16),
                 mxz.astype(bf16), last3.astype(bf16),
                 jnp.zeros((NB, tl, _C), bf16)], axis=1)
            Rp = Rn
        else:
            out_ref[...] = jnp.concatenate([er3, mxz, last3], axis=1)


def _head_body(fd_ref, fm_ref, w0_ref, b0_ref, w1_ref, b1_ref,
               wa_ref, wb_ref, bo1_ref, w2_ref, bo2_ref, out_ref, *, TB):
    f32 = jnp.float32
    f = jnp.concatenate([fd_ref[...], fm_ref[...]], axis=0)
    h = jnp.tanh(jnp.dot(f.astype(jnp.bfloat16), w0_ref[...],
                         preferred_element_type=f32) + b0_ref[...])
    h = jnp.tanh(jnp.dot(h.astype(jnp.bfloat16), w1_ref[...],
                         preferred_element_type=f32) + b1_ref[...])
    o = (jnp.dot(h[:TB].astype(jnp.bfloat16), wa_ref[...],
                 preferred_element_type=f32)
         + jnp.dot(h[TB:].astype(jnp.bfloat16), wb_ref[...],
                   preferred_element_type=f32)
         + bo1_ref[...])
    out_ref[...] = jnp.dot(o, w2_ref[...],
                           preferred_element_type=f32) + bo2_ref[...]


def kernel(data, meta, dw_w, dw_b, s1, t1, pw_w, pw_b, s2, t2,
           w0_eff, b0, w1, b1, wo1a, wo1b, bo1, wo2, bo2):
    args = (data, meta, dw_w, dw_b, s1, t1, pw_w, pw_b, s2, t2,
            w0_eff, b0, w1, b1, wo1a, wo1b, bo1, wo2, bo2)
    devs = jax.devices()
    n_dev = 2 if (len(devs) >= 2 and data.shape[0] % 512 == 0) else 1
    if n_dev == 1:
        return _impl(*args)
    mesh = Mesh(np.asarray(devs[:n_dev]), ('b',))
    specs = (P('b'), P('b')) + (P(),) * 17
    return jax.shard_map(_impl, mesh=mesh, in_specs=specs,
                         out_specs=P('b'), check_vma=False)(*args)


def _impl(data, meta, dw_w, dw_b, s1, t1, pw_w, pw_b, s2, t2,
          w0_eff, b0, w1, b1, wo1a, wo1b, bo1, wo2, bo2):
    f32 = jnp.float32
    bf16 = jnp.bfloat16
    N, _, L0 = data.shape
    B = 2 * N
    lengths = [L0]
    L = L0
    for _ in range(_NBLK):
        L = (L + 2) // 2 + 1
        lengths.append(L)
    T = lengths[-1]

    x_all = jnp.concatenate([data, meta], axis=0).reshape(B, L0).astype(f32)

    k0 = _relu(dw_b[0]) * s1[0] + t1[0]
    cb = (jnp.dot(k0, pw_w[0]) - k0[0, 0] * pw_w[0, 0:1, :] + pw_b[0])
    wv = pw_w[0, 0:1, :]
    edge0 = _relu(pw_b[0]) * s2[0] + t2[0]
    gsel = (wv * s2[0] >= 0).astype(f32)
    srow = jnp.pad(
        jnp.stack([dw_w[0, 0, 0], dw_w[0, 1, 0], dw_w[0, 2, 0],
                   dw_b[0, 0, 0], s1[0, 0, 0], t1[0, 0, 0]]).reshape(1, 6),
        ((0, 0), (0, _C - 6)))
    c0 = jnp.concatenate(
        [wv, cb, s2[0], t2[0], gsel, edge0, srow, jnp.zeros((1, _C), f32)],
        axis=0)

    dwW = dw_w[1:].astype(bf16)
    db8 = dw_b[1:].astype(bf16)
    W2 = (jnp.swapaxes(s1[1:], 1, 2) * pw_w[1:]).astype(bf16)
    b2 = jnp.einsum('boc,bcd->bod', t1[1:], pw_w[1:]) + pw_b[1:]
    edge8 = _relu(pw_b[1:]) * s2[1:] + t2[1:]
    s28 = s2[1:]
    t28 = t2[1:]

    NB = 16
    while B % NB:
        NB //= 2
    R1 = _rup16(lengths[1] + 2)

    def _w(a):
        return pl.BlockSpec(a.shape, lambda i, nd=a.ndim: (0,) * nd)

    body = functools.partial(_conv_body, NB=NB, L0=L0, lengths=tuple(lengths))
    feats = pl.pallas_call(
        body,
        out_shape=jax.ShapeDtypeStruct((B, T, _C), f32),
        grid=(B // NB,),
        in_specs=[
            pl.BlockSpec((NB, L0), lambda i: (i, 0)),
            _w(c0), _w(dwW), _w(db8), _w(W2), _w(b2),
            _w(s28), _w(t28), _w(edge8),
        ],
        out_specs=pl.BlockSpec((NB, T, _C), lambda i: (i, 0, 0)),
        scratch_shapes=[pltpu.VMEM((NB, R1, _C), bf16),
                        pltpu.VMEM((L0, NB), f32),
                        pltpu.VMEM((NB, R1, _C), f32)],
        compiler_params=pltpu.CompilerParams(
            dimension_semantics=("parallel",),
            vmem_limit_bytes=64 * 1024 * 1024),
    )(x_all, c0, dwW, db8, W2, b2, s28, t28, edge8)

    feat2d = feats.reshape(B, T * _C)

    TB = 256 if N % 256 == 0 else N
    w0b = w0_eff.astype(bf16)
    w1b = w1.astype(bf16)
    wab = wo1a.astype(bf16)
    wbb = wo1b.astype(bf16)
    hbody = functools.partial(_head_body, TB=TB)
    out = pl.pallas_call(
        hbody,
        out_shape=jax.ShapeDtypeStruct((N, wo2.shape[1]), f32),
        grid=(N // TB,),
        in_specs=[
            pl.BlockSpec((TB, T * _C), lambda i: (i, 0)),
            pl.BlockSpec((TB, T * _C), lambda i, o=N // TB: (i + o, 0)),
            _w(w0b), _w(b0), _w(w1b), _w(b1),
            _w(wab), _w(wbb), _w(bo1), _w(wo2), _w(bo2),
        ],
        out_specs=pl.BlockSpec((TB, wo2.shape[1]), lambda i: (i, 0)),
        compiler_params=pltpu.CompilerParams(
            dimension_semantics=("parallel",)),
    )(feat2d, feat2d, w0b, b0, w1b, b1, wab, wbb, bo1, wo2, bo2)
    return out
